# baseline (speedup 1.0000x reference)
import functools

import jax
import jax.numpy as jnp
from jax.experimental import pallas as pl
from jax.experimental.pallas import tpu as pltpu

_PB = 4
_CHUNKS = 2


def _body(a_ref, b2_ref, out_ref, stash_ref, mn_ref, mx_ref, *, nsteps, h, wcc):
    t = pl.program_id(2)

    @pl.when(t == 0)
    def _load_first():
        a = a_ref[:, 0, 0]
        s0 = b2_ref[0, 0, 0] - a[0]
        rest = a[1:].reshape((_PB - 1) * h, wcc)
        stash_ref[0] = s0
        stash_ref[pl.ds(1, _PB - 1)] = rest.reshape(_PB - 1, h, wcc)
        mn_ref[...] = jnp.minimum(
            jnp.min(s0, axis=0, keepdims=True), jnp.min(rest, axis=0, keepdims=True)
        )
        mx_ref[...] = jnp.maximum(
            jnp.max(s0, axis=0, keepdims=True), jnp.max(rest, axis=0, keepdims=True)
        )

    @pl.when(jnp.logical_and(t > 0, t < nsteps))
    def _load():
        a = a_ref[:, 0, 0].reshape(_PB * h, wcc)
        stash_ref[pl.ds(t * _PB, _PB)] = a.reshape(_PB, h, wcc)
        mn_ref[...] = jnp.minimum(mn_ref[...], jnp.min(a, axis=0, keepdims=True))
        mx_ref[...] = jnp.maximum(mx_ref[...], jnp.max(a, axis=0, keepdims=True))

    @pl.when(t >= nsteps)
    def _store():
        s = stash_ref[pl.ds((t - nsteps) * _PB, _PB)]
        mn = mn_ref[...]
        rng = mx_ref[...] - mn
        inv = 1.0 / jnp.where(rng == 0.0, 1.0, rng)
        out_ref[:, 0] = (s - mn) * inv


def kernel(x):
    bs, c, n, h, w = x.shape
    nb = bs * c
    cores = 2
    wcc = w // (cores * _CHUNKS)
    nsteps = nb // _PB

    body = functools.partial(_body, nsteps=nsteps, h=h, wcc=wcc)
    out = pl.pallas_call(
        body,
        grid=(cores, _CHUNKS, 2 * nsteps),
        in_specs=[
            pl.BlockSpec(
                (_PB, 1, 1, h, wcc),
                lambda i, cidx, t: (jnp.minimum(t, nsteps - 1), 0, 0, 0,
                                    i * _CHUNKS + cidx),
            ),
            pl.BlockSpec(
                (1, 1, 1, h, wcc),
                lambda i, cidx, t: (0, 0, 2, 0, i * _CHUNKS + cidx),
            ),
        ],
        out_specs=pl.BlockSpec(
            (_PB, 1, h, wcc),
            lambda i, cidx, t: (jnp.maximum(t - nsteps, 0), 0,
                                0, i * _CHUNKS + cidx),
        ),
        out_shape=jax.ShapeDtypeStruct((nb, 1, h, w), x.dtype),
        scratch_shapes=[
            pltpu.VMEM((nb, h, wcc), jnp.float32),
            pltpu.VMEM((1, wcc), jnp.float32),
            pltpu.VMEM((1, wcc), jnp.float32),
        ],
        compiler_params=pltpu.CompilerParams(
            dimension_semantics=("parallel", "arbitrary", "arbitrary"),
            vmem_limit_bytes=56 * 1024 * 1024,
        ),
    )(x, x)
    return out.reshape(bs, c, h, w)


# speedup vs baseline: 1.1057x; 1.1057x over previous
import functools

import jax
import jax.numpy as jnp
from jax.experimental import pallas as pl
from jax.experimental.pallas import tpu as pltpu

_PB = 4
_CHUNKS = 1


def _body(a_ref, b2_ref, out_ref, stash_ref, mn_ref, mx_ref, *, nsteps, h, wcc):
    t = pl.program_id(2)

    @pl.when(t == 0)
    def _load_first():
        a = a_ref[:, 0, 0]
        s0 = b2_ref[0, 0, 0] - a[0]
        rest = a[1:].reshape((_PB - 1) * h, wcc)
        stash_ref[0] = s0.astype(jnp.bfloat16)
        stash_ref[pl.ds(1, _PB - 1)] = rest.reshape(_PB - 1, h, wcc).astype(
            jnp.bfloat16
        )
        mn_ref[...] = jnp.minimum(
            jnp.min(s0, axis=0, keepdims=True), jnp.min(rest, axis=0, keepdims=True)
        )
        mx_ref[...] = jnp.maximum(
            jnp.max(s0, axis=0, keepdims=True), jnp.max(rest, axis=0, keepdims=True)
        )

    @pl.when(jnp.logical_and(t > 0, t < nsteps))
    def _load():
        a = a_ref[:, 0, 0].reshape(_PB * h, wcc)
        stash_ref[pl.ds(t * _PB, _PB)] = a.reshape(_PB, h, wcc).astype(jnp.bfloat16)
        mn_ref[...] = jnp.minimum(mn_ref[...], jnp.min(a, axis=0, keepdims=True))
        mx_ref[...] = jnp.maximum(mx_ref[...], jnp.max(a, axis=0, keepdims=True))

    @pl.when(t >= nsteps)
    def _store():
        s = stash_ref[pl.ds((t - nsteps) * _PB, _PB)].astype(jnp.float32)
        mn = mn_ref[...]
        rng = mx_ref[...] - mn
        inv = 1.0 / jnp.where(rng == 0.0, 1.0, rng)
        out_ref[:, 0] = (s - mn) * inv


def kernel(x):
    bs, c, n, h, w = x.shape
    nb = bs * c
    cores = 2
    wcc = w // (cores * _CHUNKS)
    nsteps = nb // _PB

    body = functools.partial(_body, nsteps=nsteps, h=h, wcc=wcc)
    out = pl.pallas_call(
        body,
        grid=(cores, _CHUNKS, 2 * nsteps),
        in_specs=[
            pl.BlockSpec(
                (_PB, 1, 1, h, wcc),
                lambda i, cidx, t: (jnp.minimum(t, nsteps - 1), 0, 0, 0,
                                    i * _CHUNKS + cidx),
            ),
            pl.BlockSpec(
                (1, 1, 1, h, wcc),
                lambda i, cidx, t: (0, 0, 2, 0, i * _CHUNKS + cidx),
            ),
        ],
        out_specs=pl.BlockSpec(
            (_PB, 1, h, wcc),
            lambda i, cidx, t: (jnp.maximum(t - nsteps, 0), 0,
                                0, i * _CHUNKS + cidx),
        ),
        out_shape=jax.ShapeDtypeStruct((nb, 1, h, w), x.dtype),
        scratch_shapes=[
            pltpu.VMEM((nb, h, wcc), jnp.bfloat16),
            pltpu.VMEM((1, wcc), jnp.float32),
            pltpu.VMEM((1, wcc), jnp.float32),
        ],
        compiler_params=pltpu.CompilerParams(
            dimension_semantics=("parallel", "arbitrary", "arbitrary"),
            vmem_limit_bytes=56 * 1024 * 1024,
        ),
    )(x, x)
    return out.reshape(bs, c, h, w)


# speedup vs baseline: 1.1187x; 1.0118x over previous
import functools

import jax
import jax.numpy as jnp
from jax.experimental import pallas as pl
from jax.experimental.pallas import tpu as pltpu

_GP = 4


def _body(x_ref, out_ref, stash_ref, b2_ref, stage_ref, mn_ref, mx_ref,
          in_sems, b2_sem, out_sems, *, nb, h, wc):
    i = pl.program_id(0)
    col0 = i * wc
    ng = nb // _GP

    def in_copy(g):
        return pltpu.make_async_copy(
            x_ref.at[pl.ds(g * _GP, _GP), 0, 0, :, pl.ds(col0, wc)],
            stash_ref.at[pl.ds(g * _GP, _GP)], in_sems.at[g])

    def out_copy(g):
        return pltpu.make_async_copy(
            stage_ref.at[jax.lax.rem(g, 2)],
            out_ref.at[pl.ds(g * _GP, _GP), 0, :, pl.ds(col0, wc)],
            out_sems.at[jax.lax.rem(g, 2)])

    b2_copy = pltpu.make_async_copy(
        x_ref.at[0, 0, 2, :, pl.ds(col0, wc)], b2_ref, b2_sem)
    b2_copy.start()

    def _issue(g, _):
        in_copy(g).start()
        return ()

    jax.lax.fori_loop(0, ng, _issue, (), unroll=True)

    in_copy(0).wait()
    b2_copy.wait()
    s0 = b2_ref[...] - stash_ref[0]
    stash_ref[0] = s0
    rest = stash_ref[pl.ds(1, _GP - 1)].reshape((_GP - 1) * h, wc)
    mn_ref[...] = jnp.minimum(
        jnp.min(s0, axis=0, keepdims=True), jnp.min(rest, axis=0, keepdims=True))
    mx_ref[...] = jnp.maximum(
        jnp.max(s0, axis=0, keepdims=True), jnp.max(rest, axis=0, keepdims=True))

    def _reduce(g, _):
        in_copy(g).wait()
        s = stash_ref[pl.ds(g * _GP, _GP)].reshape(_GP * h, wc)
        mn_ref[...] = jnp.minimum(mn_ref[...], jnp.min(s, axis=0, keepdims=True))
        mx_ref[...] = jnp.maximum(mx_ref[...], jnp.max(s, axis=0, keepdims=True))
        return ()

    jax.lax.fori_loop(1, ng, _reduce, ())

    mn = mn_ref[...]
    rng = mx_ref[...] - mn
    inv = 1.0 / jnp.where(rng == 0.0, 1.0, rng)

    def _store(g, _):
        @pl.when(g >= 2)
        def _():
            out_copy(g - 2).wait()

        s = stash_ref[pl.ds(g * _GP, _GP)].reshape(_GP * h, wc)
        stage_ref[jax.lax.rem(g, 2)] = ((s - mn) * inv).reshape(_GP, h, wc)
        out_copy(g).start()
        return ()

    jax.lax.fori_loop(0, ng, _store, ())
    out_copy(ng - 2).wait()
    out_copy(ng - 1).wait()


def kernel(x):
    bs, c, n, h, w = x.shape
    nb = bs * c
    cores = 2
    wc = w // cores

    body = functools.partial(_body, nb=nb, h=h, wc=wc)
    out = pl.pallas_call(
        body,
        grid=(cores,),
        in_specs=[pl.BlockSpec(memory_space=pl.ANY)],
        out_specs=pl.BlockSpec(memory_space=pl.ANY),
        out_shape=jax.ShapeDtypeStruct((nb, 1, h, w), x.dtype),
        scratch_shapes=[
            pltpu.VMEM((nb, h, wc), jnp.float32),
            pltpu.VMEM((h, wc), jnp.float32),
            pltpu.VMEM((2, _GP, h, wc), jnp.float32),
            pltpu.VMEM((1, wc), jnp.float32),
            pltpu.VMEM((1, wc), jnp.float32),
            pltpu.SemaphoreType.DMA((nb // _GP,)),
            pltpu.SemaphoreType.DMA,
            pltpu.SemaphoreType.DMA((2,)),
        ],
        compiler_params=pltpu.CompilerParams(
            dimension_semantics=("parallel",),
            vmem_limit_bytes=56 * 1024 * 1024,
        ),
    )(x)
    return out.reshape(bs, c, h, w)


# speedup vs baseline: 1.2111x; 1.0826x over previous
import functools

import jax
import jax.numpy as jnp
from jax.experimental import pallas as pl
from jax.experimental.pallas import tpu as pltpu

_GP = 4
_CH = 2


def _body(x_ref, out_ref, stash_ref, b2_ref, stage_ref, mn_ref, mx_ref,
          in_sems, b2_sems, out_sems, *, nb, h, wcc):
    i = pl.program_id(0)
    ng = nb // _GP
    base = i * _CH * wcc

    def in_copy(c, g):
        return pltpu.make_async_copy(
            x_ref.at[pl.ds(g * _GP, _GP), 0, 0, :, pl.ds(base + c * wcc, wcc)],
            stash_ref.at[c, pl.ds(g * _GP, _GP)], in_sems.at[c, g])

    def b2_copy(c):
        return pltpu.make_async_copy(
            x_ref.at[0, 0, 2, :, pl.ds(base + c * wcc, wcc)], b2_ref.at[c],
            b2_sems.at[c])

    def out_copy(k):
        c_k = jax.lax.div(k, ng)
        g_k = jax.lax.rem(k, ng)
        return pltpu.make_async_copy(
            stage_ref.at[jax.lax.rem(k, 2)],
            out_ref.at[pl.ds(g_k * _GP, _GP), 0, :,
                       pl.ds(base + c_k * wcc, wcc)],
            out_sems.at[jax.lax.rem(k, 2)])

    for c in range(_CH):
        b2_copy(c).start()

        def _issue(g, _, c=c):
            in_copy(c, g).start()
            return ()

        jax.lax.fori_loop(0, ng, _issue, (), unroll=True)

    for c in range(_CH):
        in_copy(c, 0).wait()
        b2_copy(c).wait()
        s0 = b2_ref[c] - stash_ref[c, 0]
        stash_ref[c, 0] = s0
        rest = stash_ref[c, pl.ds(1, _GP - 1)].reshape((_GP - 1) * h, wcc)
        mn_ref[...] = jnp.minimum(
            jnp.min(s0, axis=0, keepdims=True),
            jnp.min(rest, axis=0, keepdims=True))
        mx_ref[...] = jnp.maximum(
            jnp.max(s0, axis=0, keepdims=True),
            jnp.max(rest, axis=0, keepdims=True))

        def _reduce(g, _, c=c):
            in_copy(c, g).wait()
            s = stash_ref[c, pl.ds(g * _GP, _GP)].reshape(_GP * h, wcc)
            mn_ref[...] = jnp.minimum(
                mn_ref[...], jnp.min(s, axis=0, keepdims=True))
            mx_ref[...] = jnp.maximum(
                mx_ref[...], jnp.max(s, axis=0, keepdims=True))
            return ()

        jax.lax.fori_loop(1, ng, _reduce, ())

        mn = mn_ref[...]
        rng = mx_ref[...] - mn
        inv = 1.0 / jnp.where(rng == 0.0, 1.0, rng)

        def _store(g, _, c=c):
            k = c * ng + g

            @pl.when(k >= 2)
            def _():
                out_copy(k - 2).wait()

            s = stash_ref[c, pl.ds(g * _GP, _GP)].reshape(_GP * h, wcc)
            stage_ref[jax.lax.rem(k, 2)] = ((s - mn) * inv).reshape(_GP, h, wcc)
            out_copy(k).start(priority=1)
            return ()

        jax.lax.fori_loop(0, ng, _store, ())

    out_copy(_CH * ng - 2).wait()
    out_copy(_CH * ng - 1).wait()


def kernel(x):
    bs, c, n, h, w = x.shape
    nb = bs * c
    cores = 2
    wcc = w // (cores * _CH)

    body = functools.partial(_body, nb=nb, h=h, wcc=wcc)
    out = pl.pallas_call(
        body,
        grid=(cores,),
        in_specs=[pl.BlockSpec(memory_space=pl.ANY)],
        out_specs=pl.BlockSpec(memory_space=pl.ANY),
        out_shape=jax.ShapeDtypeStruct((nb, 1, h, w), x.dtype),
        scratch_shapes=[
            pltpu.VMEM((_CH, nb, h, wcc), jnp.float32),
            pltpu.VMEM((_CH, h, wcc), jnp.float32),
            pltpu.VMEM((2, _GP, h, wcc), jnp.float32),
            pltpu.VMEM((1, wcc), jnp.float32),
            pltpu.VMEM((1, wcc), jnp.float32),
            pltpu.SemaphoreType.DMA((_CH, nb // _GP)),
            pltpu.SemaphoreType.DMA((_CH,)),
            pltpu.SemaphoreType.DMA((2,)),
        ],
        compiler_params=pltpu.CompilerParams(
            dimension_semantics=("parallel",),
            vmem_limit_bytes=56 * 1024 * 1024,
        ),
    )(x)
    return out.reshape(bs, c, h, w)
